# Initial kernel scaffold; baseline (speedup 1.0000x reference)
import jax
import jax.numpy as jnp
from jax.experimental import pallas as pl


def kernel(x, tau):
    raise NotImplementedError("write your pallas kernel here")


# speedup vs baseline: 27.8064x; 27.8064x over previous
import functools

import jax
import jax.numpy as jnp
from jax.experimental import pallas as pl
from jax.experimental.pallas import tpu as pltpu

_HW = 1024
_K = 256
_BLOCK_R = 256


def _topk_mask_body(tau_ref, x_ref, o_ref):
    x = x_ref[...]
    u = jax.lax.bitcast_convert_type(x, jnp.int32) & jnp.int32(0x7FFFFFFF)
    t = jnp.zeros((x.shape[0], 1), jnp.int32)
    for b in range(30, -1, -1):
        cand = t | jnp.int32(1 << b)
        cnt = jnp.sum((u >= cand).astype(jnp.int32), axis=1, keepdims=True)
        t = jnp.where(cnt >= _K, cand, t)
    sparse = jnp.where(u >= t, x, jnp.float32(0.0))
    tau = tau_ref[0]
    o_ref[...] = sparse * tau + x * (jnp.float32(1.0) - tau)


@jax.jit
def kernel(x, tau):
    n, c, h, w = x.shape
    rows = n * c
    x2 = x.reshape(rows, h * w)
    tau_arr = jnp.asarray(tau, jnp.float32).reshape(1)
    grid = rows // _BLOCK_R
    out = pl.pallas_call(
        _topk_mask_body,
        grid=(grid,),
        in_specs=[
            pl.BlockSpec(memory_space=pltpu.MemorySpace.SMEM),
            pl.BlockSpec((_BLOCK_R, _HW), lambda i: (i, 0)),
        ],
        out_specs=pl.BlockSpec((_BLOCK_R, _HW), lambda i: (i, 0)),
        out_shape=jax.ShapeDtypeStruct((rows, h * w), jnp.float32),
    )(tau_arr, x2)
    return out.reshape(n, c, h, w)
